# baseline (speedup 1.0000x reference)
import functools

import jax
import jax.numpy as jnp
from jax import lax
from jax.experimental import pallas as pl
from jax.experimental.pallas import tpu as pltpu
from jax.experimental.pallas import tpu_sc as plsc

N_GROUPS_ = 8
GROUP_SIZE_ = 8
N_EXPERTS_ = 64
TOPK_GROUPS_ = 4
TOPK_ = 2
T_ = 8192
NW_ = 32
TPW_ = T_ // NW_
NTILES_ = TPW_ // 16


def _mm_kernel(x_ref, w_ref, s_out_ref):
    s_out_ref[...] = jax.lax.dot_general(
        w_ref[...],
        x_ref[...],
        (((1,), (1,)), ((), ())),
        preferred_element_type=jnp.float32,
    )


def _scores_t(x, W):
    T, D = x.shape
    bt = 1024
    return pl.pallas_call(
        _mm_kernel,
        grid=(T // bt,),
        in_specs=[
            pl.BlockSpec((bt, D), lambda i: (i, 0)),
            pl.BlockSpec((N_EXPERTS_, D), lambda i: (0, 0)),
        ],
        out_specs=pl.BlockSpec((N_EXPERTS_, bt), lambda i: (0, i)),
        out_shape=jax.ShapeDtypeStruct((N_EXPERTS_, T), jnp.float32),
        compiler_params=pltpu.CompilerParams(
            dimension_semantics=("arbitrary",),
        ),
    )(x, W)


def _route_body(st_hbm, w_out, i_out, sbuf, wbuf, ibuf):
    cid = lax.axis_index("c")
    sid = lax.axis_index("s")
    wid = sid * 2 + cid
    base = wid * TPW_

    pltpu.sync_copy(st_hbm.at[:, pl.ds(base, TPW_)], sbuf)

    def tile(j, _):
        off = j * 16

        gms = []
        for g in range(N_GROUPS_):
            gm = sbuf[g * GROUP_SIZE_, pl.ds(off, 16)]
            for k in range(1, GROUP_SIZE_):
                gm = jnp.maximum(gm, sbuf[g * GROUP_SIZE_ + k, pl.ds(off, 16)])
            gms.append(gm)
        m = gms[0]
        for g in range(1, N_GROUPS_):
            m = jnp.maximum(m, gms[g])

        denom = jnp.zeros((16,), jnp.float32)
        for e in range(N_EXPERTS_):
            ee = jnp.exp(sbuf[e, pl.ds(off, 16)] - m)
            denom = denom + ee
            sbuf[e, pl.ds(off, 16)] = ee

        pgms = [jnp.exp(g_ - m) / denom for g_ in gms]

        one = jnp.ones((16,), jnp.int32)
        zero = jnp.zeros((16,), jnp.int32)
        sels = []
        for g in range(N_GROUPS_):
            cnt = zero
            for h in range(N_GROUPS_):
                if h == g:
                    continue
                beats = (pgms[h] >= pgms[g]) if h < g else (pgms[h] > pgms[g])
                cnt = cnt + jnp.where(beats, one, zero)
            sels.append(cnt < TOPK_GROUPS_)

        neg = jnp.full((16,), -1.0, jnp.float32)
        v1 = neg
        v2 = neg
        i1 = jnp.zeros((16,), jnp.int32)
        i2 = jnp.zeros((16,), jnp.int32)
        for g in range(N_GROUPS_):
            for k in range(GROUP_SIZE_):
                e = g * GROUP_SIZE_ + k
                pe = sbuf[e, pl.ds(off, 16)] / denom
                pm = jnp.where(sels[g], pe, neg)
                eid = jnp.full((16,), e, jnp.int32)
                gt1 = pm > v1
                gt2 = pm > v2
                v2 = jnp.where(gt1, v1, jnp.where(gt2, pm, v2))
                i2 = jnp.where(gt1, i1, jnp.where(gt2, eid, i2))
                v1 = jnp.where(gt1, pm, v1)
                i1 = jnp.where(gt1, eid, i1)

        wbuf[0, pl.ds(off, 16)] = v1
        wbuf[1, pl.ds(off, 16)] = v2
        ibuf[0, pl.ds(off, 16)] = i1
        ibuf[1, pl.ds(off, 16)] = i2
        return _

    lax.fori_loop(0, NTILES_, tile, None)

    pltpu.sync_copy(wbuf, w_out.at[:, pl.ds(base, TPW_)])
    pltpu.sync_copy(ibuf, i_out.at[:, pl.ds(base, TPW_)])


@jax.jit
def kernel(x, W):
    st = _scores_t(x, W)
    route = pl.kernel(
        _route_body,
        mesh=plsc.VectorSubcoreMesh(core_axis_name="c", subcore_axis_name="s"),
        out_type=[
            jax.ShapeDtypeStruct((TOPK_, T_), jnp.float32),
            jax.ShapeDtypeStruct((TOPK_, T_), jnp.int32),
        ],
        scratch_types=[
            pltpu.VMEM((N_EXPERTS_, TPW_), jnp.float32),
            pltpu.VMEM((TOPK_, TPW_), jnp.float32),
            pltpu.VMEM((TOPK_, TPW_), jnp.int32),
        ],
    )
    weights_t, indices_t = route(st)
    return weights_t.T.astype(x.dtype), indices_t.T


# speedup vs baseline: 1.8677x; 1.8677x over previous
import functools

import jax
import jax.numpy as jnp
from jax.experimental import pallas as pl
from jax.experimental.pallas import tpu as pltpu

N_GROUPS_ = 8
GROUP_SIZE_ = 8
N_EXPERTS_ = 64
TOPK_GROUPS_ = 4
TOPK_ = 2
NEG_INF_ = float("-inf")


def _gate_kernel(xa_ref, xb_ref, w_ref, w_out_ref, i_out_ref):
    sub = xa_ref.shape[0]
    for h, xr in enumerate((xa_ref, xb_ref)):
        _gate_subtile(
            xr[...],
            w_ref[...],
            w_out_ref.at[:, pl.ds(h * sub, sub)],
            i_out_ref.at[:, pl.ds(h * sub, sub)],
        )


def _gate_subtile(x, w, w_out_ref, i_out_ref):
    st = jax.lax.dot_general(
        w,
        x,
        (((1,), (1,)), ((), ())),
        preferred_element_type=jnp.float32,
    )
    bt = st.shape[1]

    row_max = jnp.max(st, axis=0, keepdims=True)
    e = jnp.exp(st - row_max)
    p = e / jnp.sum(e, axis=0, keepdims=True)

    gms = [
        jnp.max(p[g * GROUP_SIZE_ : (g + 1) * GROUP_SIZE_], axis=0, keepdims=True)
        for g in range(N_GROUPS_)
    ]

    sels = []
    for g in range(N_GROUPS_):
        cnt = None
        for h in range(N_GROUPS_):
            if h == g:
                continue
            if h < g:
                beats = gms[h] >= gms[g]
            else:
                beats = gms[h] > gms[g]
            b = beats.astype(jnp.int32)
            cnt = b if cnt is None else cnt + b
        sels.append(cnt < TOPK_GROUPS_)

    masked = jnp.concatenate(
        [
            jnp.where(
                sels[g], p[g * GROUP_SIZE_ : (g + 1) * GROUP_SIZE_], NEG_INF_
            )
            for g in range(N_GROUPS_)
        ],
        axis=0,
    )

    expert_id = jax.lax.broadcasted_iota(jnp.int32, (N_EXPERTS_, bt), 0)

    ws = []
    idxs = []
    for _ in range(TOPK_):
        vmax = jnp.max(masked, axis=0, keepdims=True)
        cand = jnp.where(masked == vmax, expert_id, N_EXPERTS_)
        win = jnp.min(cand, axis=0, keepdims=True)
        ws.append(vmax)
        idxs.append(win)
        masked = jnp.where(expert_id == win, NEG_INF_, masked)

    w_out_ref[...] = jnp.concatenate(ws, axis=0)
    i_out_ref[...] = jnp.concatenate(idxs, axis=0)


@jax.jit
def kernel(x, W):
    T, D = x.shape
    bt = 1024
    grid = (T // bt,)
    weights_t, indices_t = pl.pallas_call(
        _gate_kernel,
        grid=grid,
        in_specs=[
            pl.BlockSpec((bt // 2, D), lambda i: (2 * i, 0)),
            pl.BlockSpec((bt // 2, D), lambda i: (2 * i + 1, 0)),
            pl.BlockSpec((N_EXPERTS_, D), lambda i: (0, 0)),
        ],
        out_specs=[
            pl.BlockSpec((TOPK_, bt), lambda i: (0, i)),
            pl.BlockSpec((TOPK_, bt), lambda i: (0, i)),
        ],
        out_shape=[
            jax.ShapeDtypeStruct((TOPK_, T), jnp.float32),
            jax.ShapeDtypeStruct((TOPK_, T), jnp.int32),
        ],
        compiler_params=pltpu.CompilerParams(
            dimension_semantics=("arbitrary",),
        ),
    )(x, x, W)
    return weights_t.T.astype(x.dtype), indices_t.T
